# baseline (speedup 1.0000x reference)
import functools

import jax
import jax.numpy as jnp
from jax import lax
from jax.experimental import pallas as pl
from jax.experimental.pallas import tpu as pltpu
from jax.experimental.pallas import tpu_sc as plsc

_NC = 2
_NS = 16
_NW = _NC * _NS
_LANES = 16


def _sc_gather(cat3, table_t):
    nf, dim, vp = table_t.shape
    _, ng, _ = cat3.shape
    nplanes = nf * dim
    per_t = nplanes // _NW
    qg = ng // 4

    mesh = plsc.VectorSubcoreMesh(core_axis_name="c", subcore_axis_name="s")

    @functools.partial(
        pl.kernel,
        mesh=mesh,
        out_type=jax.ShapeDtypeStruct((nplanes, ng, 128), jnp.float32),
        compiler_params=pltpu.CompilerParams(
            use_tc_tiling_on_sc=False, needs_layout_passes=False
        ),
        scratch_types=[
            pltpu.VMEM((vp,), jnp.float32),
            pltpu.VMEM((ng, 128), jnp.int32),
            pltpu.VMEM((2, qg, 128), jnp.float32),
            pltpu.SemaphoreType.DMA,
            pltpu.SemaphoreType.DMA,
            pltpu.SemaphoreType.DMA,
            pltpu.SemaphoreType.DMA,
        ],
    )
    def k(cat_hbm, table_hbm, out_hbm, plane_v, catv, outv, sem_p, sem_c,
          sem_o0, sem_o1):
        wid = lax.axis_index("s") * _NC + lax.axis_index("c")
        sem_o = (sem_o0, sem_o1)

        def plane(pi, _):
            p = wid * per_t + pi
            fi = lax.div(p, dim)
            di = lax.rem(p, dim)
            cp_p = pltpu.async_copy(table_hbm.at[fi, di], plane_v, sem_p)
            cp_c = pltpu.async_copy(cat_hbm.at[fi], catv, sem_c)
            cp_p.wait()
            cp_c.wait()
            for q in range(4):
                buf = q % 2

                def drain():
                    pltpu.make_async_copy(
                        outv.at[buf], out_hbm.at[p, pl.ds(q * qg, qg)],
                        sem_o[buf],
                    ).wait()

                if q >= 2:
                    drain()
                else:
                    pl.when(pi > 0)(drain)

                @plsc.parallel_loop(0, qg, unroll=4)
                def _(r):
                    for cc in range(8):
                        idx = catv[q * qg + r, pl.ds(cc * _LANES, _LANES)]
                        outv[buf, r, pl.ds(cc * _LANES, _LANES)] = (
                            plsc.load_gather(plane_v, [idx])
                        )

                pltpu.async_copy(
                    outv.at[buf], out_hbm.at[p, pl.ds(q * qg, qg)], sem_o[buf]
                )
            return 0

        lax.fori_loop(0, per_t, plane, 0)
        for buf in range(2):
            pltpu.make_async_copy(
                outv.at[buf], out_hbm.at[0, pl.ds(0, qg)], sem_o[buf]
            ).wait()

    return k(cat3, table_t)


def _mlp2_body(emb_ref, dense_ref, mean_ref, var_ref, w1e_ref, w1d_ref,
               b1_ref, woutt_ref, bout_ref, out_ref):
    normed = (dense_ref[...] - mean_ref[...]) * lax.rsqrt(var_ref[...] + 1e-6)
    hd = jnp.dot(normed, w1d_ref[...], preferred_element_type=jnp.float32)
    dn = (((0,), (0,)), ((), ()))
    for rb in range(8):
        h = lax.dot_general(emb_ref[:, rb, :], w1e_ref[...], dn,
                            preferred_element_type=jnp.float32)
        h = jnp.maximum(h + hd[rb * 128:(rb + 1) * 128, :] + b1_ref[...], 0.0)
        o = jnp.sum(h * woutt_ref[...], axis=1, keepdims=True) + bout_ref[...]
        out_ref[pl.ds(rb * 128, 128), :] = jax.nn.sigmoid(o)


def _tc_mlp2(emb3, dense, mean, var, w1e, w1d, b1, woutt, bout):
    npl = emb3.shape[0]
    bsz, nd = dense.shape
    hid = w1d.shape[1]
    bm = 1024
    gb = bm // 128
    grid = (bsz // bm,)
    return pl.pallas_call(
        _mlp2_body,
        grid=grid,
        in_specs=[
            pl.BlockSpec((npl, gb, 128), lambda i: (0, i, 0)),
            pl.BlockSpec((bm, nd), lambda i: (i, 0)),
            pl.BlockSpec((1, nd), lambda i: (0, 0)),
            pl.BlockSpec((1, nd), lambda i: (0, 0)),
            pl.BlockSpec((npl, hid), lambda i: (0, 0)),
            pl.BlockSpec((nd, hid), lambda i: (0, 0)),
            pl.BlockSpec((1, hid), lambda i: (0, 0)),
            pl.BlockSpec((1, hid), lambda i: (0, 0)),
            pl.BlockSpec((1, 1), lambda i: (0, 0)),
        ],
        out_specs=pl.BlockSpec((bm, 1), lambda i: (i, 0)),
        out_shape=jax.ShapeDtypeStruct((bsz, 1), jnp.float32),
    )(emb3, dense, mean, var, w1e, w1d, b1, woutt, bout)


def kernel(cat_indices, dense_features, emb_tables, norm_mean, norm_var, W1,
           b1, W_out, b_out):
    b, f = cat_indices.shape
    _, v, d = emb_tables.shape
    vp = (v + 7) // 8 * 8
    cat3 = cat_indices.T.reshape(f, b // 128, 128)
    table_t = jnp.pad(
        emb_tables, ((0, 0), (0, vp - v), (0, 0))
    ).transpose(0, 2, 1)

    emb3 = _sc_gather(cat3, table_t)

    out = _tc_mlp2(
        emb3,
        dense_features,
        norm_mean.reshape(1, -1),
        norm_var.reshape(1, -1),
        W1[: f * d],
        W1[f * d:],
        b1.reshape(1, -1),
        W_out.reshape(1, -1),
        b_out.reshape(1, 1),
    )
    return out


# speedup vs baseline: 1.0022x; 1.0022x over previous
import functools

import jax
import jax.numpy as jnp
from jax import lax
from jax.experimental import pallas as pl
from jax.experimental.pallas import tpu as pltpu
from jax.experimental.pallas import tpu_sc as plsc

_NC = 2
_NS = 16
_NW = _NC * _NS
_LANES = 16


def _sc_gather(cat3, table_t):
    nf, dim, vp = table_t.shape
    _, ng, _ = cat3.shape
    nplanes = nf * dim
    per_t = nplanes // _NW
    qg = ng // 4

    mesh = plsc.VectorSubcoreMesh(core_axis_name="c", subcore_axis_name="s")

    @functools.partial(
        pl.kernel,
        mesh=mesh,
        out_type=jax.ShapeDtypeStruct((nplanes, ng, 128), jnp.float32),
        compiler_params=pltpu.CompilerParams(
            use_tc_tiling_on_sc=False, needs_layout_passes=False
        ),
        scratch_types=[
            pltpu.VMEM((vp,), jnp.float32),
            pltpu.VMEM((ng, 128), jnp.int32),
            pltpu.VMEM((2, qg, 128), jnp.float32),
            pltpu.SemaphoreType.DMA,
            pltpu.SemaphoreType.DMA,
            pltpu.SemaphoreType.DMA,
            pltpu.SemaphoreType.DMA,
        ],
    )
    def k(cat_hbm, table_hbm, out_hbm, plane_v, catv, outv, sem_p, sem_c,
          sem_o0, sem_o1):
        wid = lax.axis_index("s") * _NC + lax.axis_index("c")
        sem_o = (sem_o0, sem_o1)

        def plane(pi, _):
            p = wid * per_t + pi
            fi = lax.div(p, dim)
            di = lax.rem(p, dim)
            cp_p = pltpu.async_copy(table_hbm.at[fi, di], plane_v, sem_p)
            cp_c = pltpu.async_copy(cat_hbm.at[fi], catv, sem_c)
            cp_p.wait()
            cp_c.wait()
            for q in range(4):
                buf = q % 2

                def drain():
                    pltpu.make_async_copy(
                        outv.at[buf], out_hbm.at[p, pl.ds(q * qg, qg)],
                        sem_o[buf],
                    ).wait()

                if q >= 2:
                    drain()
                else:
                    pl.when(pi > 0)(drain)

                @plsc.parallel_loop(0, qg, unroll=4)
                def _(r):
                    for cc in range(8):
                        idx = catv[q * qg + r, pl.ds(cc * _LANES, _LANES)]
                        outv[buf, r, pl.ds(cc * _LANES, _LANES)] = (
                            plsc.load_gather(plane_v, [idx])
                        )

                pltpu.async_copy(
                    outv.at[buf], out_hbm.at[p, pl.ds(q * qg, qg)], sem_o[buf]
                )
            return 0

        lax.fori_loop(0, per_t, plane, 0)
        for buf in range(2):
            pltpu.make_async_copy(
                outv.at[buf], out_hbm.at[0, pl.ds(0, qg)], sem_o[buf]
            ).wait()

    return k(cat3, table_t)


def _mlp2_body(emb_ref, dense_ref, mean_ref, var_ref, w1e_ref, w1d_ref,
               b1_ref, woutt_ref, bout_ref, out_ref):
    normed = (dense_ref[...] - mean_ref[...]) * lax.rsqrt(var_ref[...] + 1e-6)
    hd = jnp.dot(normed, w1d_ref[...], preferred_element_type=jnp.float32)
    dn = (((0,), (0,)), ((), ()))
    for rb in range(8):
        h = lax.dot_general(emb_ref[:, rb, :], w1e_ref[...], dn,
                            preferred_element_type=jnp.float32)
        h = jnp.maximum(h + hd[rb * 128:(rb + 1) * 128, :] + b1_ref[...], 0.0)
        o = jnp.sum(h * woutt_ref[...], axis=1, keepdims=True) + bout_ref[...]
        out_ref[pl.ds(rb * 128, 128), :] = jax.nn.sigmoid(o)


def _tc_mlp2(emb3, dense, mean, var, w1e, w1d, b1, woutt, bout):
    npl = emb3.shape[0]
    bsz, nd = dense.shape
    hid = w1d.shape[1]
    bm = 1024
    gb = bm // 128
    grid = (bsz // bm,)
    return pl.pallas_call(
        _mlp2_body,
        grid=grid,
        in_specs=[
            pl.BlockSpec((npl, gb, 128), lambda i: (0, i, 0)),
            pl.BlockSpec((bm, nd), lambda i: (i, 0)),
            pl.BlockSpec((1, nd), lambda i: (0, 0)),
            pl.BlockSpec((1, nd), lambda i: (0, 0)),
            pl.BlockSpec((npl, hid), lambda i: (0, 0)),
            pl.BlockSpec((nd, hid), lambda i: (0, 0)),
            pl.BlockSpec((1, hid), lambda i: (0, 0)),
            pl.BlockSpec((1, hid), lambda i: (0, 0)),
            pl.BlockSpec((1, 1), lambda i: (0, 0)),
        ],
        out_specs=pl.BlockSpec((bm, 1), lambda i: (i, 0)),
        out_shape=jax.ShapeDtypeStruct((bsz, 1), jnp.float32),
    )(emb3, dense, mean, var, w1e, w1d, b1, woutt, bout)


def kernel(cat_indices, dense_features, emb_tables, norm_mean, norm_var, W1,
           b1, W_out, b_out):
    b, f = cat_indices.shape
    _, v, d = emb_tables.shape
    vp = (v + 15) // 16 * 16
    cat3 = cat_indices.T.reshape(f, b // 128, 128)
    table_t = jnp.pad(
        emb_tables, ((0, 0), (0, vp - v), (0, 0))
    ).transpose(0, 2, 1)

    emb3 = _sc_gather(cat3, table_t)

    out = _tc_mlp2(
        emb3,
        dense_features,
        norm_mean.reshape(1, -1),
        norm_var.reshape(1, -1),
        W1[: f * d],
        W1[f * d:],
        b1.reshape(1, -1),
        W_out.reshape(1, -1),
        b_out.reshape(1, 1),
    )
    return out


# speedup vs baseline: 1.0448x; 1.0426x over previous
import functools

import jax
import jax.numpy as jnp
from jax import lax
from jax.experimental import pallas as pl
from jax.experimental.pallas import tpu as pltpu
from jax.experimental.pallas import tpu_sc as plsc

_NC = 2
_NS = 16
_NW = _NC * _NS
_LANES = 16


def _sc_gather(cat3, table_r, vocab):
    nf, dh, _ = table_r.shape
    dim = dh * 8
    _, ng, _ = cat3.shape
    nplanes = nf * dim
    vp = (vocab + 7) // 8 * 8
    per_t = nplanes // _NW
    qg = ng // 4

    mesh = plsc.VectorSubcoreMesh(core_axis_name="c", subcore_axis_name="s")

    @functools.partial(
        pl.kernel,
        mesh=mesh,
        out_type=jax.ShapeDtypeStruct((nplanes, ng, 128), jnp.float32),
        compiler_params=pltpu.CompilerParams(
            use_tc_tiling_on_sc=False, needs_layout_passes=False
        ),
        scratch_types=[
            pltpu.VMEM((vp,), jnp.float32),
            pltpu.VMEM((ng, 128), jnp.int32),
            pltpu.VMEM((2, qg, 128), jnp.float32),
            pltpu.SemaphoreType.DMA,
            pltpu.SemaphoreType.DMA,
            pltpu.SemaphoreType.DMA,
            pltpu.SemaphoreType.DMA,
        ],
    )
    def k(cat_hbm, table_hbm, out_hbm, plane_v, catv, outv, sem_p, sem_c,
          sem_o0, sem_o1):
        wid = lax.axis_index("s") * _NC + lax.axis_index("c")
        sem_o = (sem_o0, sem_o1)

        def plane(pi, _):
            p = wid * per_t + pi
            fi = lax.div(p, dim)
            di = lax.rem(p, dim)
            hh = lax.div(di, 8)
            start = lax.rem(di, 8) * vocab
            sa = lax.bitwise_and(start, ~7)
            delta = start - sa
            cp_p = pltpu.async_copy(
                table_hbm.at[fi, hh, pl.ds(pl.multiple_of(sa, 8), vp)],
                plane_v, sem_p,
            )
            cp_c = pltpu.async_copy(cat_hbm.at[fi], catv, sem_c)
            cp_p.wait()
            cp_c.wait()
            for q in range(4):
                buf = q % 2

                def drain():
                    pltpu.make_async_copy(
                        outv.at[buf], out_hbm.at[p, pl.ds(q * qg, qg)],
                        sem_o[buf],
                    ).wait()

                if q >= 2:
                    drain()
                else:
                    pl.when(pi > 0)(drain)

                @plsc.parallel_loop(0, qg, unroll=4)
                def _(r):
                    for cc in range(8):
                        idx = catv[q * qg + r, pl.ds(cc * _LANES, _LANES)]
                        outv[buf, r, pl.ds(cc * _LANES, _LANES)] = (
                            plsc.load_gather(plane_v, [idx + delta])
                        )

                pltpu.async_copy(
                    outv.at[buf], out_hbm.at[p, pl.ds(q * qg, qg)], sem_o[buf]
                )
            return 0

        lax.fori_loop(0, per_t, plane, 0)
        for buf in range(2):
            pltpu.make_async_copy(
                outv.at[buf], out_hbm.at[0, pl.ds(0, qg)], sem_o[buf]
            ).wait()

    return k(cat3, table_r)


def _mlp2_body(emb_ref, dense_ref, mean_ref, var_ref, w1e_ref, w1d_ref,
               b1_ref, woutt_ref, bout_ref, out_ref):
    normed = (dense_ref[...] - mean_ref[...]) * lax.rsqrt(var_ref[...] + 1e-6)
    hd = jnp.dot(normed, w1d_ref[...], preferred_element_type=jnp.float32)
    dn = (((0,), (0,)), ((), ()))
    for rb in range(8):
        h = lax.dot_general(emb_ref[:, rb, :], w1e_ref[...], dn,
                            preferred_element_type=jnp.float32)
        h = jnp.maximum(h + hd[rb * 128:(rb + 1) * 128, :] + b1_ref[...], 0.0)
        o = jnp.sum(h * woutt_ref[...], axis=1, keepdims=True) + bout_ref[...]
        out_ref[pl.ds(rb * 128, 128), :] = jax.nn.sigmoid(o)


def _tc_mlp2(emb3, dense, mean, var, w1e, w1d, b1, woutt, bout):
    npl = emb3.shape[0]
    bsz, nd = dense.shape
    hid = w1d.shape[1]
    bm = 1024
    gb = bm // 128
    grid = (bsz // bm,)
    return pl.pallas_call(
        _mlp2_body,
        grid=grid,
        in_specs=[
            pl.BlockSpec((npl, gb, 128), lambda i: (0, i, 0)),
            pl.BlockSpec((bm, nd), lambda i: (i, 0)),
            pl.BlockSpec((1, nd), lambda i: (0, 0)),
            pl.BlockSpec((1, nd), lambda i: (0, 0)),
            pl.BlockSpec((npl, hid), lambda i: (0, 0)),
            pl.BlockSpec((nd, hid), lambda i: (0, 0)),
            pl.BlockSpec((1, hid), lambda i: (0, 0)),
            pl.BlockSpec((1, hid), lambda i: (0, 0)),
            pl.BlockSpec((1, 1), lambda i: (0, 0)),
        ],
        out_specs=pl.BlockSpec((bm, 1), lambda i: (i, 0)),
        out_shape=jax.ShapeDtypeStruct((bsz, 1), jnp.float32),
    )(emb3, dense, mean, var, w1e, w1d, b1, woutt, bout)


def kernel(cat_indices, dense_features, emb_tables, norm_mean, norm_var, W1,
           b1, W_out, b_out):
    b, f = cat_indices.shape
    _, v, d = emb_tables.shape
    cat3 = cat_indices.T.reshape(f, b // 128, 128)
    table_r = emb_tables.transpose(0, 2, 1).reshape(f, d // 8, 8 * v)

    emb3 = _sc_gather(cat3, table_r, v)

    out = _tc_mlp2(
        emb3,
        dense_features,
        norm_mean.reshape(1, -1),
        norm_var.reshape(1, -1),
        W1[: f * d],
        W1[f * d:],
        b1.reshape(1, -1),
        W_out.reshape(1, -1),
        b_out.reshape(1, 1),
    )
    return out


# speedup vs baseline: 5.8888x; 5.6363x over previous
import functools

import jax
import jax.numpy as jnp
from jax import lax
from jax.experimental import pallas as pl
from jax.experimental.pallas import tpu as pltpu
from jax.experimental.pallas import tpu_sc as plsc

_NC = 2
_NS = 16
_NW = _NC * _NS
_LANES = 16


def _sc_gather(cat3, table4, dim):
    nplanes, vb, _ = table4.shape
    nf, ng, _ = cat3.shape
    per_t = nplanes // _NW
    qg = ng // 4

    mesh = plsc.VectorSubcoreMesh(core_axis_name="c", subcore_axis_name="s")

    @functools.partial(
        pl.kernel,
        mesh=mesh,
        out_type=jax.ShapeDtypeStruct((nplanes, ng, 128), jnp.float32),
        compiler_params=pltpu.CompilerParams(
            use_tc_tiling_on_sc=False, needs_layout_passes=False
        ),
        scratch_types=[
            pltpu.VMEM((vb, 128), jnp.float32),
            pltpu.VMEM((ng, 128), jnp.int32),
            pltpu.VMEM((2, qg, 128), jnp.float32),
            pltpu.SemaphoreType.DMA,
            pltpu.SemaphoreType.DMA,
            pltpu.SemaphoreType.DMA,
            pltpu.SemaphoreType.DMA,
        ],
    )
    def k(cat_hbm, table_hbm, out_hbm, plane_v, catv, outv, sem_p, sem_c,
          sem_o0, sem_o1):
        wid = lax.axis_index("s") * _NC + lax.axis_index("c")
        sem_o = (sem_o0, sem_o1)

        def plane(pi, _):
            p = wid * per_t + pi
            fi = lax.div(p, dim)
            cp_p = pltpu.async_copy(table_hbm.at[p], plane_v, sem_p)
            cp_c = pltpu.async_copy(cat_hbm.at[fi], catv, sem_c)
            cp_p.wait()
            cp_c.wait()
            for q in range(4):
                buf = q % 2

                def drain():
                    pltpu.make_async_copy(
                        outv.at[buf], out_hbm.at[p, pl.ds(q * qg, qg)],
                        sem_o[buf],
                    ).wait()

                if q >= 2:
                    drain()
                else:
                    pl.when(pi > 0)(drain)

                @plsc.parallel_loop(0, qg, unroll=4)
                def _(r):
                    for cc in range(8):
                        idx = catv[q * qg + r, pl.ds(cc * _LANES, _LANES)]
                        hi = lax.shift_right_logical(idx, 7)
                        lo = lax.bitwise_and(idx, 127)
                        outv[buf, r, pl.ds(cc * _LANES, _LANES)] = (
                            plsc.load_gather(plane_v, [hi, lo])
                        )

                pltpu.async_copy(
                    outv.at[buf], out_hbm.at[p, pl.ds(q * qg, qg)], sem_o[buf]
                )
            return 0

        lax.fori_loop(0, per_t, plane, 0)
        for buf in range(2):
            pltpu.make_async_copy(
                outv.at[buf], out_hbm.at[0, pl.ds(0, qg)], sem_o[buf]
            ).wait()

    return k(cat3, table4)


def _mlp2_body(emb_ref, dense_ref, mean_ref, var_ref, w1e_ref, w1d_ref,
               b1_ref, woutt_ref, bout_ref, out_ref):
    normed = (dense_ref[...] - mean_ref[...]) * lax.rsqrt(var_ref[...] + 1e-6)
    hd = jnp.dot(normed, w1d_ref[...], preferred_element_type=jnp.float32)
    dn = (((0,), (0,)), ((), ()))
    for rb in range(8):
        h = lax.dot_general(emb_ref[:, rb, :], w1e_ref[...], dn,
                            preferred_element_type=jnp.float32)
        h = jnp.maximum(h + hd[rb * 128:(rb + 1) * 128, :] + b1_ref[...], 0.0)
        o = jnp.sum(h * woutt_ref[...], axis=1, keepdims=True) + bout_ref[...]
        out_ref[pl.ds(rb * 128, 128), :] = jax.nn.sigmoid(o)


def _tc_mlp2(emb3, dense, mean, var, w1e, w1d, b1, woutt, bout):
    npl = emb3.shape[0]
    bsz, nd = dense.shape
    hid = w1d.shape[1]
    bm = 1024
    gb = bm // 128
    grid = (bsz // bm,)
    return pl.pallas_call(
        _mlp2_body,
        grid=grid,
        in_specs=[
            pl.BlockSpec((npl, gb, 128), lambda i: (0, i, 0)),
            pl.BlockSpec((bm, nd), lambda i: (i, 0)),
            pl.BlockSpec((1, nd), lambda i: (0, 0)),
            pl.BlockSpec((1, nd), lambda i: (0, 0)),
            pl.BlockSpec((npl, hid), lambda i: (0, 0)),
            pl.BlockSpec((nd, hid), lambda i: (0, 0)),
            pl.BlockSpec((1, hid), lambda i: (0, 0)),
            pl.BlockSpec((1, hid), lambda i: (0, 0)),
            pl.BlockSpec((1, 1), lambda i: (0, 0)),
        ],
        out_specs=pl.BlockSpec((bm, 1), lambda i: (i, 0)),
        out_shape=jax.ShapeDtypeStruct((bsz, 1), jnp.float32),
    )(emb3, dense, mean, var, w1e, w1d, b1, woutt, bout)


def kernel(cat_indices, dense_features, emb_tables, norm_mean, norm_var, W1,
           b1, W_out, b_out):
    b, f = cat_indices.shape
    _, v, d = emb_tables.shape
    vb = (v + 127) // 128
    cat3 = cat_indices.T.reshape(f, b // 128, 128)
    table4 = jnp.pad(
        emb_tables, ((0, 0), (0, vb * 128 - v), (0, 0))
    ).transpose(0, 2, 1).reshape(f * d, vb, 128)

    emb3 = _sc_gather(cat3, table4, d)

    out = _tc_mlp2(
        emb3,
        dense_features,
        norm_mean.reshape(1, -1),
        norm_var.reshape(1, -1),
        W1[: f * d],
        W1[f * d:],
        b1.reshape(1, -1),
        W_out.reshape(1, -1),
        b_out.reshape(1, 1),
    )
    return out
